# baseline (speedup 1.0000x reference)
import functools

import jax
import jax.numpy as jnp
from jax import lax
from jax.experimental import pallas as pl
from jax.experimental.pallas import tpu as pltpu
from jax.experimental.pallas import tpu_sc as plsc

N = 10000
E = 320000
D = 128
H = 256

_K = 5
_EC = E // _K


_NW = 32
_EPW = _EC // _NW
_CH = 80
_NCH = _EPW // _CH


def _sc_gather_build():
    mesh = plsc.VectorSubcoreMesh(core_axis_name="c", subcore_axis_name="s")

    @functools.partial(
        pl.kernel,
        mesh=mesh,
        out_type=[
            jax.ShapeDtypeStruct((_EC, D), jnp.float32),
            jax.ShapeDtypeStruct((_EC, D), jnp.float32),
        ],
        scratch_types=[
            pltpu.VMEM((_EPW,), jnp.int32),
            pltpu.VMEM((_EPW,), jnp.int32),
            pltpu.VMEM((2, _CH, D), jnp.float32),
            pltpu.VMEM((2, _CH, D), jnp.float32),
            pltpu.SemaphoreType.DMA,
            pltpu.SemaphoreType.DMA,
        ],
    )
    def sc_gather(nf_hbm, src_hbm, dst_hbm, out_s_hbm, out_d_hbm,
                  idx_s, idx_d, rows_s, rows_d, sem_s, sem_d):
        wid = lax.axis_index("s") * 2 + lax.axis_index("c")
        base = wid * _EPW
        pltpu.sync_copy(src_hbm.at[pl.ds(base, _EPW)], idx_s)
        pltpu.sync_copy(dst_hbm.at[pl.ds(base, _EPW)], idx_d)

        def fire(j, slot):
            off = j * _CH
            pltpu.async_copy(nf_hbm.at[idx_s.at[pl.ds(off, _CH)]],
                             rows_s.at[slot], sem_s)
            pltpu.async_copy(nf_hbm.at[idx_d.at[pl.ds(off, _CH)]],
                             rows_d.at[slot], sem_d)

        def drain(j, slot):
            off = j * _CH
            pltpu.make_async_copy(nf_hbm.at[idx_s.at[pl.ds(off, _CH)]],
                                  rows_s.at[slot], sem_s).wait()
            pltpu.sync_copy(rows_s.at[slot],
                            out_s_hbm.at[pl.ds(base + off, _CH)])
            pltpu.make_async_copy(nf_hbm.at[idx_d.at[pl.ds(off, _CH)]],
                                  rows_d.at[slot], sem_d).wait()
            pltpu.sync_copy(rows_d.at[slot],
                            out_d_hbm.at[pl.ds(base + off, _CH)])

        fire(0, 0)

        def body(j, carry):
            slot = lax.rem(j, 2)

            @pl.when(j + 1 < _NCH)
            def _():
                fire(j + 1, 1 - slot)

            drain(j, slot)
            return carry

        lax.fori_loop(0, _NCH, body, 0)

    return sc_gather


_BE = 1280


def _mlp_body(ef_ref, gs_ref, gd_ref, w1t_ref, b1_ref,
              w2t_ref, b2_ref, gamma_ref, beta_ref, out_ref):
    ef = ef_ref[...]
    x = jnp.concatenate(
        [ef, gs_ref[...], gd_ref[...]], axis=-1).astype(jnp.bfloat16)
    h = jnp.dot(x, w1t_ref[...], preferred_element_type=jnp.float32)
    h += b1_ref[...]
    h = h * jax.nn.sigmoid(h)
    y = jnp.dot(h.astype(jnp.bfloat16), w2t_ref[...],
                preferred_element_type=jnp.float32)
    y += b2_ref[...]
    mu = jnp.mean(y, axis=-1, keepdims=True)
    var = jnp.mean((y - mu) ** 2, axis=-1, keepdims=True)
    y = (y - mu) * lax.rsqrt(var + 1e-5) * gamma_ref[...] + beta_ref[...]
    out_ref[...] = y + ef


def _mlp_call(ef, gs, gd, W1T, b1, W2T, b2, gamma, beta, interpret=False):
    grid = (_EC // _BE,)
    eb = pl.BlockSpec((_BE, D), lambda i: (i, 0))
    full = lambda shape: pl.BlockSpec(shape, lambda i: tuple(0 for _ in shape))
    return pl.pallas_call(
        _mlp_body,
        grid=grid,
        in_specs=[
            eb, eb, eb,
            full((3 * D, H)), full((1, H)),
            full((H, D)), full((1, D)), full((1, D)), full((1, D)),
        ],
        out_specs=eb,
        out_shape=jax.ShapeDtypeStruct((_EC, D), jnp.float32),
        interpret=interpret,
    )(ef, gs, gd, W1T, b1, W2T, b2, gamma, beta)


def kernel(edge_feats, node_feats, edge_index, We, Ws, Wd, b1, W2, b2,
           gamma, beta):
    src = edge_index[0].astype(jnp.int32)
    dst = edge_index[1].astype(jnp.int32)
    W1T = jnp.concatenate([We.T, Ws.T, Wd.T], axis=0).astype(jnp.bfloat16)
    b1r = b1.reshape(1, H)
    W2T = W2.T.astype(jnp.bfloat16)
    b2r = b2.reshape(1, D)
    gr = gamma.reshape(1, D)
    br = beta.reshape(1, D)

    sc_gather = _sc_gather_build()
    outs = []
    for c in range(_K):
        sl = slice(c * _EC, (c + 1) * _EC)
        gs, gd = sc_gather(node_feats, src[sl], dst[sl])
        outs.append(_mlp_call(edge_feats[sl], gs, gd,
                              W1T, b1r, W2T, b2r, gr, br))
    out = jnp.concatenate(outs, axis=0)
    return (out, node_feats)


# speedup vs baseline: 1.4539x; 1.4539x over previous
import functools

import jax
import jax.numpy as jnp
from jax import lax
from jax.experimental import pallas as pl
from jax.experimental.pallas import tpu as pltpu
from jax.experimental.pallas import tpu_sc as plsc

N = 10000
E = 320000
D = 128
H = 256

_K = 1
_EC = E // _K


_NW = 32
_EPW = _EC // _NW
_CH = 80
_NCH = _EPW // _CH


def _sc_gather_build():
    mesh = plsc.VectorSubcoreMesh(core_axis_name="c", subcore_axis_name="s")

    @functools.partial(
        pl.kernel,
        mesh=mesh,
        out_type=[
            jax.ShapeDtypeStruct((_EC, D), jnp.float32),
            jax.ShapeDtypeStruct((_EC, D), jnp.float32),
        ],
        scratch_types=[
            pltpu.VMEM((_EPW,), jnp.int32),
            pltpu.VMEM((_EPW,), jnp.int32),
            pltpu.VMEM((2, _CH, D), jnp.float32),
            pltpu.VMEM((2, _CH, D), jnp.float32),
            pltpu.SemaphoreType.DMA,
            pltpu.SemaphoreType.DMA,
        ],
    )
    def sc_gather(nf_hbm, src_hbm, dst_hbm, out_s_hbm, out_d_hbm,
                  idx_s, idx_d, rows_s, rows_d, sem_s, sem_d):
        wid = lax.axis_index("s") * 2 + lax.axis_index("c")
        base = wid * _EPW
        pltpu.sync_copy(src_hbm.at[pl.ds(base, _EPW)], idx_s)
        pltpu.sync_copy(dst_hbm.at[pl.ds(base, _EPW)], idx_d)

        def fire(j, slot):
            off = j * _CH
            pltpu.async_copy(nf_hbm.at[idx_s.at[pl.ds(off, _CH)]],
                             rows_s.at[slot], sem_s)
            pltpu.async_copy(nf_hbm.at[idx_d.at[pl.ds(off, _CH)]],
                             rows_d.at[slot], sem_d)

        def drain(j, slot):
            off = j * _CH
            pltpu.make_async_copy(nf_hbm.at[idx_s.at[pl.ds(off, _CH)]],
                                  rows_s.at[slot], sem_s).wait()
            pltpu.sync_copy(rows_s.at[slot],
                            out_s_hbm.at[pl.ds(base + off, _CH)])
            pltpu.make_async_copy(nf_hbm.at[idx_d.at[pl.ds(off, _CH)]],
                                  rows_d.at[slot], sem_d).wait()
            pltpu.sync_copy(rows_d.at[slot],
                            out_d_hbm.at[pl.ds(base + off, _CH)])

        fire(0, 0)

        def body(j, carry):
            slot = lax.rem(j, 2)

            @pl.when(j + 1 < _NCH)
            def _():
                fire(j + 1, 1 - slot)

            drain(j, slot)
            return carry

        lax.fori_loop(0, _NCH, body, 0)

    return sc_gather


_BE = 2560


def _mlp_body(ef_ref, gs_ref, gd_ref, w1t_ref, b1_ref,
              w2t_ref, b2_ref, gamma_ref, beta_ref, out_ref):
    ef = ef_ref[...]
    x = jnp.concatenate(
        [ef, gs_ref[...], gd_ref[...]], axis=-1).astype(jnp.bfloat16)
    h = jnp.dot(x, w1t_ref[...], preferred_element_type=jnp.float32)
    h += b1_ref[...]
    h = (0.5 * h) * (1.0 + jnp.tanh(0.5 * h))
    y = jnp.dot(h.astype(jnp.bfloat16), w2t_ref[...],
                preferred_element_type=jnp.float32)
    y += b2_ref[...]
    mu = jnp.mean(y, axis=-1, keepdims=True)
    var = jnp.mean((y - mu) ** 2, axis=-1, keepdims=True)
    y = (y - mu) * lax.rsqrt(var + 1e-5) * gamma_ref[...] + beta_ref[...]
    out_ref[...] = y + ef


def _mlp_call(ef, gs, gd, W1T, b1, W2T, b2, gamma, beta, interpret=False):
    grid = (_EC // _BE,)
    eb = pl.BlockSpec((_BE, D), lambda i: (i, 0))
    full = lambda shape: pl.BlockSpec(shape, lambda i: tuple(0 for _ in shape))
    return pl.pallas_call(
        _mlp_body,
        grid=grid,
        in_specs=[
            eb, eb, eb,
            full((3 * D, H)), full((1, H)),
            full((H, D)), full((1, D)), full((1, D)), full((1, D)),
        ],
        out_specs=eb,
        out_shape=jax.ShapeDtypeStruct((_EC, D), jnp.float32),
        interpret=interpret,
    )(ef, gs, gd, W1T, b1, W2T, b2, gamma, beta)


def kernel(edge_feats, node_feats, edge_index, We, Ws, Wd, b1, W2, b2,
           gamma, beta):
    src = edge_index[0].astype(jnp.int32)
    dst = edge_index[1].astype(jnp.int32)
    W1T = jnp.concatenate([We.T, Ws.T, Wd.T], axis=0).astype(jnp.bfloat16)
    b1r = b1.reshape(1, H)
    W2T = W2.T.astype(jnp.bfloat16)
    b2r = b2.reshape(1, D)
    gr = gamma.reshape(1, D)
    br = beta.reshape(1, D)

    sc_gather = _sc_gather_build()
    outs = []
    for c in range(_K):
        sl = slice(c * _EC, (c + 1) * _EC)
        gs, gd = sc_gather(node_feats, src[sl], dst[sl])
        outs.append(_mlp_call(edge_feats[sl], gs, gd,
                              W1T, b1r, W2T, b2r, gr, br))
    out = jnp.concatenate(outs, axis=0)
    return (out, node_feats)


# speedup vs baseline: 1.4555x; 1.0011x over previous
import functools

import jax
import jax.numpy as jnp
from jax import lax
from jax.experimental import pallas as pl
from jax.experimental.pallas import tpu as pltpu
from jax.experimental.pallas import tpu_sc as plsc

N = 10000
E = 320000
D = 128
H = 256

_K = 1
_EC = E // _K


_NW = 32
_EPW = _EC // _NW
_CH = 80
_NCH = _EPW // _CH


def _sc_gather_build():
    mesh = plsc.VectorSubcoreMesh(core_axis_name="c", subcore_axis_name="s")

    @functools.partial(
        pl.kernel,
        mesh=mesh,
        out_type=[
            jax.ShapeDtypeStruct((_EC, D), jnp.float32),
            jax.ShapeDtypeStruct((_EC, D), jnp.float32),
        ],
        scratch_types=[
            pltpu.VMEM((_EPW,), jnp.int32),
            pltpu.VMEM((_EPW,), jnp.int32),
            pltpu.VMEM((3, _CH, D), jnp.float32),
            pltpu.VMEM((3, _CH, D), jnp.float32),
            pltpu.SemaphoreType.DMA,
            pltpu.SemaphoreType.DMA,
            pltpu.SemaphoreType.DMA,
            pltpu.SemaphoreType.DMA,
        ],
    )
    def sc_gather(nf_hbm, src_hbm, dst_hbm, out_s_hbm, out_d_hbm,
                  idx_s, idx_d, rows_s, rows_d,
                  sem_si, sem_di, sem_so, sem_do):
        wid = lax.axis_index("s") * 2 + lax.axis_index("c")
        base = wid * _EPW
        pltpu.sync_copy(src_hbm.at[pl.ds(base, _EPW)], idx_s)
        pltpu.sync_copy(dst_hbm.at[pl.ds(base, _EPW)], idx_d)

        def fire_in(j):
            off = j * _CH
            slot = lax.rem(j, 3)
            pltpu.async_copy(nf_hbm.at[idx_s.at[pl.ds(off, _CH)]],
                             rows_s.at[slot], sem_si)
            pltpu.async_copy(nf_hbm.at[idx_d.at[pl.ds(off, _CH)]],
                             rows_d.at[slot], sem_di)

        def wait_in(j):
            off = j * _CH
            slot = lax.rem(j, 3)
            pltpu.make_async_copy(nf_hbm.at[idx_s.at[pl.ds(off, _CH)]],
                                  rows_s.at[slot], sem_si).wait()
            pltpu.make_async_copy(nf_hbm.at[idx_d.at[pl.ds(off, _CH)]],
                                  rows_d.at[slot], sem_di).wait()

        def fire_out(j):
            off = j * _CH
            slot = lax.rem(j, 3)
            pltpu.async_copy(rows_s.at[slot],
                             out_s_hbm.at[pl.ds(base + off, _CH)], sem_so)
            pltpu.async_copy(rows_d.at[slot],
                             out_d_hbm.at[pl.ds(base + off, _CH)], sem_do)

        def wait_out(j):
            off = j * _CH
            slot = lax.rem(j, 3)
            pltpu.make_async_copy(rows_s.at[slot],
                                  out_s_hbm.at[pl.ds(base + off, _CH)],
                                  sem_so).wait()
            pltpu.make_async_copy(rows_d.at[slot],
                                  out_d_hbm.at[pl.ds(base + off, _CH)],
                                  sem_do).wait()

        fire_in(0)
        fire_in(1)

        def body(j, carry):
            wait_in(j)
            fire_out(j)

            @pl.when(j + 2 < _NCH)
            def _():
                @pl.when(j >= 1)
                def _():
                    wait_out(j - 1)

                fire_in(j + 2)

            return carry

        lax.fori_loop(0, _NCH, body, 0)
        wait_out(_NCH - 2)
        wait_out(_NCH - 1)

    return sc_gather


_BE = 2560


def _mlp_body(ef_ref, gs_ref, gd_ref, w1t_ref, b1_ref,
              w2t_ref, b2_ref, gamma_ref, beta_ref, out_ref):
    ef = ef_ref[...]
    x = jnp.concatenate(
        [ef, gs_ref[...], gd_ref[...]], axis=-1).astype(jnp.bfloat16)
    h = jnp.dot(x, w1t_ref[...], preferred_element_type=jnp.float32)
    h += b1_ref[...]
    h = (0.5 * h) * (1.0 + jnp.tanh(0.5 * h))
    y = jnp.dot(h.astype(jnp.bfloat16), w2t_ref[...],
                preferred_element_type=jnp.float32)
    y += b2_ref[...]
    mu = jnp.mean(y, axis=-1, keepdims=True)
    var = jnp.mean((y - mu) ** 2, axis=-1, keepdims=True)
    y = (y - mu) * lax.rsqrt(var + 1e-5) * gamma_ref[...] + beta_ref[...]
    out_ref[...] = y + ef


def _mlp_call(ef, gs, gd, W1T, b1, W2T, b2, gamma, beta, interpret=False):
    grid = (_EC // _BE,)
    eb = pl.BlockSpec((_BE, D), lambda i: (i, 0))
    full = lambda shape: pl.BlockSpec(shape, lambda i: tuple(0 for _ in shape))
    return pl.pallas_call(
        _mlp_body,
        grid=grid,
        in_specs=[
            eb, eb, eb,
            full((3 * D, H)), full((1, H)),
            full((H, D)), full((1, D)), full((1, D)), full((1, D)),
        ],
        out_specs=eb,
        out_shape=jax.ShapeDtypeStruct((_EC, D), jnp.float32),
        interpret=interpret,
    )(ef, gs, gd, W1T, b1, W2T, b2, gamma, beta)


def kernel(edge_feats, node_feats, edge_index, We, Ws, Wd, b1, W2, b2,
           gamma, beta):
    src = edge_index[0].astype(jnp.int32)
    dst = edge_index[1].astype(jnp.int32)
    W1T = jnp.concatenate([We.T, Ws.T, Wd.T], axis=0).astype(jnp.bfloat16)
    b1r = b1.reshape(1, H)
    W2T = W2.T.astype(jnp.bfloat16)
    b2r = b2.reshape(1, D)
    gr = gamma.reshape(1, D)
    br = beta.reshape(1, D)

    sc_gather = _sc_gather_build()
    outs = []
    for c in range(_K):
        sl = slice(c * _EC, (c + 1) * _EC)
        gs, gd = sc_gather(node_feats, src[sl], dst[sl])
        outs.append(_mlp_call(edge_feats[sl], gs, gd,
                              W1T, b1r, W2T, b2r, gr, br))
    out = jnp.concatenate(outs, axis=0)
    return (out, node_feats)
